# baseline (speedup 1.0000x reference)
import functools

import jax
import jax.numpy as jnp
from jax import lax
from jax.experimental import pallas as pl
from jax.experimental.pallas import tpu as pltpu
from jax.experimental.pallas import tpu_sc as plsc

NBUF = 2
L = 16


@functools.lru_cache(maxsize=None)
def _build_lookup(batch: int, hist: int, vocab: int, embed_dim: int):
    info = plsc.get_sparse_core_info()
    num_workers = info.num_cores * info.num_subcores
    bpw = batch // num_workers
    assert batch % num_workers == 0 and bpw == 128 and embed_dim == 64
    assert hist % NBUF == 0

    mesh = plsc.VectorSubcoreMesh(core_axis_name="c", subcore_axis_name="s")

    @functools.partial(
        pl.kernel,
        mesh=mesh,
        compiler_params=pltpu.CompilerParams(
            use_tc_tiling_on_sc=False, needs_layout_passes=False
        ),
        out_type=jax.ShapeDtypeStruct(
            (hist, embed_dim // 8, num_workers, 8, bpw), jnp.float32
        ),
        scratch_types=[
            pltpu.VMEM((bpw, hist), jnp.int32),
            pltpu.VMEM((hist, bpw), jnp.int32),
        ]
        + [pltpu.VMEM((bpw, embed_dim), jnp.float32)] * NBUF
        + [pltpu.VMEM((embed_dim // 8, 8, bpw), jnp.float32)] * NBUF
        + [pltpu.SemaphoreType.DMA] * (2 * NBUF),
    )
    def lookup(tok_hbm, table_hbm, out_hbm, idx_v, idxt_v, *bufs):
        rows_v = bufs[:NBUF]
        blk_v = bufs[NBUF : 2 * NBUF]
        sem_g = bufs[2 * NBUF : 3 * NBUF]
        sem_w = bufs[3 * NBUF :]
        wid = lax.axis_index("s") * info.num_cores + lax.axis_index("c")
        b0 = wid * bpw
        pltpu.sync_copy(tok_hbm.at[pl.ds(b0, bpw)], idx_v)

        lanes = lax.iota(jnp.int32, L)

        def build_idxt(h, carry):
            for g in range(bpw // L):
                col = jnp.full((L,), h, jnp.int32)
                vals = plsc.load_gather(idx_v, [g * L + lanes, col])
                idxt_v[h, pl.ds(g * L, L)] = vals
            return carry

        lax.fori_loop(0, hist, build_idxt, 0)

        def start_gather(h, buf):
            pltpu.async_copy(table_hbm.at[idxt_v.at[h]], rows_v[buf], sem_g[buf])

        def start_write(h, buf):
            pltpu.async_copy(blk_v[buf], out_hbm.at[h, :, wid], sem_w[buf])

        def transpose_block(buf):
            for fb in range(embed_dim // 8):
                for fl in range(8):
                    for g in range(bpw // L):
                        col = jnp.full((L,), 8 * fb + fl, jnp.int32)
                        vals = plsc.load_gather(rows_v[buf], [g * L + lanes, col])
                        blk_v[buf][fb, fl, pl.ds(g * L, L)] = vals

        def drain_g(buf):
            pltpu.make_async_copy(
                table_hbm.at[pl.ds(0, bpw)], rows_v[buf], sem_g[buf]
            ).wait()

        def drain_w(buf):
            pltpu.make_async_copy(
                blk_v[buf], out_hbm.at[0, :, wid], sem_w[buf]
            ).wait()

        for b in range(NBUF):
            start_gather(b, b)

        n_groups = hist // NBUF

        def group(g, carry):
            for b in range(NBUF):
                h = g * NBUF + b
                drain_g(b)

                @pl.when(g > 0)
                def _():
                    drain_w(b)

                transpose_block(b)
                start_write(h, b)

                @pl.when(h + NBUF < hist)
                def _():
                    start_gather(h + NBUF, b)
            return carry

        lax.fori_loop(0, n_groups, group, 0)

        for b in range(NBUF):
            drain_w(b)

    return lookup


def kernel(tokens, token_embed_weight):
    batch, hist = tokens.shape
    vocab, embed_dim = token_embed_weight.shape
    lookup = _build_lookup(batch, hist, vocab, embed_dim)
    out5d = lookup(tokens.astype(jnp.int32), token_embed_weight)
    out = out5d.transpose(2, 4, 0, 1, 3).reshape(batch, hist, embed_dim)
    return out


# speedup vs baseline: 2.0212x; 2.0212x over previous
import functools

import jax
import jax.numpy as jnp
from jax import lax
from jax.experimental import pallas as pl
from jax.experimental.pallas import tpu as pltpu
from jax.experimental.pallas import tpu_sc as plsc

NBUF = 2
SPLIT = (128, 72)
PAD = 128


@functools.lru_cache(maxsize=None)
def _build_lookup(batch: int, hist: int, vocab: int, embed_dim: int):
    info = plsc.get_sparse_core_info()
    num_workers = info.num_cores * info.num_subcores
    bpw = batch // num_workers
    n_groups = bpw // NBUF
    assert batch % num_workers == 0 and bpw % NBUF == 0
    assert sum(SPLIT) == hist

    mesh = plsc.VectorSubcoreMesh(core_axis_name="c", subcore_axis_name="s")

    @functools.partial(
        pl.kernel,
        mesh=mesh,
        compiler_params=pltpu.CompilerParams(use_tc_tiling_on_sc=False),
        out_type=jax.ShapeDtypeStruct((batch, hist, PAD), jnp.float32),
        scratch_types=[
            pltpu.VMEM((bpw, hist), jnp.int32),
        ]
        + [pltpu.VMEM((hist, PAD), jnp.float32)] * NBUF
        + [pltpu.SemaphoreType.DMA] * (2 * NBUF),
    )
    def lookup(tok_hbm, table_hbm, out_hbm, idx_v, *bufs):
        rows_v = bufs[:NBUF]
        sem_g = bufs[NBUF : 2 * NBUF]
        sem_w = bufs[2 * NBUF :]
        wid = lax.axis_index("s") * info.num_cores + lax.axis_index("c")
        b0 = wid * bpw
        pltpu.sync_copy(tok_hbm.at[pl.ds(b0, bpw)], idx_v)

        def start_gather(r, buf):
            off = 0
            for width in SPLIT:
                pltpu.async_copy(
                    table_hbm.at[idx_v.at[r, pl.ds(off, width)]],
                    rows_v[buf].at[pl.ds(off, width)],
                    sem_g[buf],
                )
                off += width

        def start_write(r, buf):
            pltpu.async_copy(rows_v[buf], out_hbm.at[b0 + r], sem_w[buf])

        def drain_g(buf):
            pltpu.make_async_copy(
                table_hbm.at[pl.ds(0, hist)], rows_v[buf], sem_g[buf]
            ).wait()

        def drain_w(buf):
            pltpu.make_async_copy(rows_v[buf], out_hbm.at[b0], sem_w[buf]).wait()

        for b in range(NBUF):
            start_gather(b, b)

        def group(g, carry):
            for b in range(NBUF):
                r = g * NBUF + b
                drain_g(b)

                @pl.when(g > 0)
                def _():
                    drain_w(b)

                start_write(r, b)

                @pl.when(r + NBUF < bpw)
                def _():
                    start_gather(r + NBUF, b)
            return carry

        lax.fori_loop(0, n_groups, group, 0)

        for b in range(NBUF):
            drain_w(b)

    return lookup


def kernel(tokens, token_embed_weight):
    batch, hist = tokens.shape
    vocab, embed_dim = token_embed_weight.shape
    tpad = jnp.pad(token_embed_weight, ((0, 0), (0, PAD - embed_dim)))
    lookup = _build_lookup(batch, hist, vocab, embed_dim)
    wide = lookup(tokens.astype(jnp.int32), tpad)
    return wide[:, :, :embed_dim]
